# baseline (speedup 1.0000x reference)
import functools

import jax
import jax.numpy as jnp
from jax import lax
from jax.experimental import pallas as pl
from jax.experimental.pallas import tpu as pltpu
from jax.experimental.pallas import tpu_sc as plsc

_NC = 2
_NS = 16


@functools.partial(jax.jit, static_argnums=())
def kernel(tokens, token_embedding, position_embedding):
    Bt, T = tokens.shape
    V, D = token_embedding.shape
    B = Bt * T
    NW = _NC * _NS
    b_per_w = B // NW
    C = T // 7
    n_chunks = b_per_w // C

    idx = tokens.reshape(NW, n_chunks, C).astype(jnp.int32)

    mesh = plsc.VectorSubcoreMesh(core_axis_name="c", subcore_axis_name="s")

    @functools.partial(
        pl.kernel,
        out_type=jax.ShapeDtypeStruct((B, D), jnp.float32),
        mesh=mesh,
        scratch_types=[
            pltpu.VMEM((n_chunks, C), jnp.int32),
            pltpu.VMEM((T, D), jnp.float32),
            pltpu.VMEM((C, D), jnp.float32),
            pltpu.SemaphoreType.DMA,
        ],
        compiler_params=pltpu.CompilerParams(use_tc_tiling_on_sc=False),
    )
    def body(idx_hbm, table_hbm, pos_hbm, out_hbm, idx_v, pos_v, buf, sem):
        wid = lax.axis_index("s") * _NC + lax.axis_index("c")
        pltpu.sync_copy(idx_hbm.at[wid], idx_v)
        pltpu.sync_copy(pos_hbm, pos_v)
        row_base = wid * b_per_w

        def chunk_body(c, carry):
            off = (c * C) % T

            def prefill_row(r, carry2):
                for j in range(D // 16):
                    buf[r, pl.ds(j * 16, 16)] = pos_v[off + r, pl.ds(j * 16, 16)]
                return carry2

            lax.fori_loop(0, C, prefill_row, 0)

            pltpu.async_copy(table_hbm.at[idx_v.at[c]], buf, sem, add=True).wait()
            pltpu.sync_copy(buf, out_hbm.at[pl.ds(row_base + c * C, C)])
            return carry

        lax.fori_loop(0, n_chunks, chunk_body, 0)

    out = body(idx, token_embedding, position_embedding)
    return out.reshape(Bt, T, D)


# speedup vs baseline: 1.8644x; 1.8644x over previous
import functools

import jax
import jax.numpy as jnp
from jax import lax
from jax.experimental import pallas as pl
from jax.experimental.pallas import tpu as pltpu
from jax.experimental.pallas import tpu_sc as plsc

_NC = 2
_NS = 16
_NBUF = 7
_LOOK = 4


def kernel(tokens, token_embedding, position_embedding):
    Bt, T = tokens.shape
    V, D = token_embedding.shape
    B = Bt * T
    NW = _NC * _NS
    b_per_w = B // NW
    C = T // 7
    n_chunks = b_per_w // C
    n_rounds = n_chunks // _NBUF
    NVEC = D // 16

    idx = tokens.reshape(NW, n_chunks, C).astype(jnp.int32)

    mesh = plsc.VectorSubcoreMesh(core_axis_name="c", subcore_axis_name="s")

    @functools.partial(
        pl.kernel,
        out_type=jax.ShapeDtypeStruct((B, D), jnp.float32),
        mesh=mesh,
        scratch_types=[
            pltpu.VMEM((n_chunks, C), jnp.int32),
            pltpu.VMEM((T, D), jnp.float32),
            [pltpu.VMEM((C, D), jnp.float32) for _ in range(_NBUF)],
            [pltpu.SemaphoreType.DMA for _ in range(_NBUF)],
            [pltpu.SemaphoreType.DMA for _ in range(_NBUF)],
        ],
        compiler_params=pltpu.CompilerParams(use_tc_tiling_on_sc=False),
    )
    def body(idx_hbm, table_hbm, pos_hbm, out_hbm,
             idx_v, pos_v, bufs, gsems, wsems):
        wid = lax.axis_index("s") * _NC + lax.axis_index("c")
        pltpu.sync_copy(idx_hbm.at[wid], idx_v)
        pltpu.sync_copy(pos_hbm, pos_v)
        row_base = wid * b_per_w

        def fire_gather(c, slot):
            pltpu.async_copy(table_hbm.at[idx_v.at[c]], bufs[slot], gsems[slot])

        def wait_gather(c, slot):
            pltpu.make_async_copy(
                table_hbm.at[idx_v.at[c]], bufs[slot], gsems[slot]).wait()

        def fire_wb(c, slot):
            pltpu.async_copy(
                bufs[slot], out_hbm.at[pl.ds(row_base + c * C, C)], wsems[slot])

        def wait_wb(slot):
            pltpu.make_async_copy(
                bufs[slot], out_hbm.at[pl.ds(row_base, C)], wsems[slot]).wait()

        def add_pos(c, slot):
            off = lax.rem(c * C, T)

            def row_fn(r, carry):
                for j in range(NVEC):
                    plsc.addupdate(bufs[slot].at[r, pl.ds(j * 16, 16)],
                                   pos_v[off + r, pl.ds(j * 16, 16)])
                return carry

            lax.fori_loop(0, C, row_fn, 0)

        def step(c, s, *, wait_w, fire_g):
            t = (s + _LOOK) % _NBUF
            if wait_w:
                wait_wb(t)
            if fire_g:
                fire_gather(c + _LOOK, t)
            wait_gather(c, s)
            add_pos(c, s)
            fire_wb(c, s)

        for c0 in range(_LOOK):
            fire_gather(c0, c0)

        for s in range(_NBUF):
            step(s, s, wait_w=(s + _LOOK >= _NBUF), fire_g=True)

        def round_body(i, carry):
            for s in range(_NBUF):
                step(i * _NBUF + s, s, wait_w=True, fire_g=True)
            return carry

        lax.fori_loop(1, n_rounds - 1, round_body, 0)

        i_last = n_rounds - 1
        for s in range(_NBUF):
            c = i_last * _NBUF + s
            step(c, s, wait_w=(c + _LOOK < n_chunks),
                 fire_g=(c + _LOOK < n_chunks))

        for s in range(_NBUF):
            wait_wb(s)

    out = body(idx, token_embedding, position_embedding)
    return out.reshape(Bt, T, D)
